# baseline (speedup 1.0000x reference)
import jax
import jax.numpy as jnp
from jax import lax
from jax.experimental import pallas as pl
from jax.experimental.pallas import tpu as pltpu
from jax.experimental.pallas import tpu_sc as plsc

B = 16384
T = 26
D = 16
BCW = 64

NC = 2
NS = 16
NW = NC * NS
BPW = B // NW
C = 64
NCHUNK = BPW // C
NG = C // 16


def _body(user_hbm, item_hbm, uemb_hbm, iemb_hbm, ubc_hbm, ibc_hbm,
          gb_hbm, gc_hbm, out_hbm,
          idx_u, idx_i, urows, irows, bcu, bci, gbv, gcv, outv, sem):
  wid = lax.axis_index("s") * NC + lax.axis_index("c")
  base = wid * BPW

  pltpu.sync_copy(user_hbm.at[pl.ds(base, BPW)], idx_u)
  pltpu.sync_copy(item_hbm.at[pl.ds(base, BPW)], idx_i)
  pltpu.sync_copy(gb_hbm, gbv)
  pltpu.sync_copy(gc_hbm, gcv)
  gb0 = gbv[pl.ds(0, 16)]
  gb1 = gbv[pl.ds(16, 16)]
  gc0 = gcv[pl.ds(0, 16)]
  gc1 = gcv[pl.ds(16, 16)]

  iota = lax.iota(jnp.int32, 16)
  rot = [jnp.bitwise_and(iota + d, 15) for d in range(D)]

  @pl.loop(0, NCHUNK)
  def _chunk(c):
    iu = idx_u.at[pl.ds(c * C, C)]
    ii = idx_i.at[pl.ds(c * C, C)]
    cps = [
        pltpu.async_copy(uemb_hbm.at[iu], urows, sem),
        pltpu.async_copy(iemb_hbm.at[ii], irows, sem),
        pltpu.async_copy(ubc_hbm.at[iu], bcu, sem),
        pltpu.async_copy(ibc_hbm.at[ii], bci, sem),
    ]
    for cp in cps:
      cp.wait()

    @pl.loop(0, NG)
    def _group(g):
      rvec = iota + g * 16
      acc = jnp.zeros((16,), jnp.float32)
      for t in range(T):
        tvec = jnp.full((16,), t, jnp.int32)
        ps = [jnp.zeros((16,), jnp.float32) for _ in range(4)]
        for d in range(D):
          uu = plsc.load_gather(urows, [rvec, tvec, rot[d]])
          iv = plsc.load_gather(irows, [rvec, tvec, rot[d]])
          ps[d % 4] = ps[d % 4] + uu * iv
        s = (ps[0] + ps[1]) + (ps[2] + ps[3])
        s = s + plsc.load_gather(bcu, [rvec, tvec])
        s = s + plsc.load_gather(bci, [rvec, tvec])
        s = s + (gb0[t] if t < 16 else gb1[t - 16])
        sig = 1.0 / (1.0 + jnp.exp(-s))
        ctvec = jnp.full((16,), T + t, jnp.int32)
        cf = plsc.load_gather(bcu, [rvec, ctvec])
        cf = cf + plsc.load_gather(bci, [rvec, ctvec])
        cf = cf + (gc0[t] if t < 16 else gc1[t - 16])
        acc = acc + sig * cf
      outv[pl.ds(c * C + g * 16, 16)] = acc

  pltpu.sync_copy(outv, out_hbm.at[pl.ds(base, BPW)])


@jax.jit
def _run(user, item, uemb, iemb, ubc, ibc, gb, gc):
  mesh = plsc.VectorSubcoreMesh(core_axis_name="c", subcore_axis_name="s")
  f = pl.kernel(
      _body,
      out_type=jax.ShapeDtypeStruct((B,), jnp.float32),
      mesh=mesh,
      scratch_types=[
          pltpu.VMEM((BPW,), jnp.int32),
          pltpu.VMEM((BPW,), jnp.int32),
          pltpu.VMEM((C, T, D), jnp.float32),
          pltpu.VMEM((C, T, D), jnp.float32),
          pltpu.VMEM((C, BCW), jnp.float32),
          pltpu.VMEM((C, BCW), jnp.float32),
          pltpu.VMEM((32,), jnp.float32),
          pltpu.VMEM((32,), jnp.float32),
          pltpu.VMEM((BPW,), jnp.float32),
          pltpu.SemaphoreType.DMA,
      ],
      compiler_params=pltpu.CompilerParams(use_tc_tiling_on_sc=False,
                                           needs_layout_passes=False),
  )
  return f(user, item, uemb, iemb, ubc, ibc, gb, gc)


def kernel(user, item, user_tag_embeddings, item_tag_embeddings,
           user_aspect_bias, item_aspect_bias, global_aspect_bias,
           user_coeff, item_coeff, global_coeff):
  user = user.astype(jnp.int32)
  item = item.astype(jnp.int32)
  pad = jnp.zeros((user_aspect_bias.shape[0], BCW - 2 * T),
                  user_aspect_bias.dtype)
  ubc = jnp.concatenate([user_aspect_bias, user_coeff, pad], axis=1)
  ibc = jnp.concatenate([item_aspect_bias, item_coeff, pad], axis=1)
  gb = jnp.pad(global_aspect_bias.reshape(T), (0, 32 - T))
  gc = jnp.pad(global_coeff.reshape(T), (0, 32 - T))
  return _run(user, item, user_tag_embeddings, item_tag_embeddings,
              ubc, ibc, gb, gc)


# speedup vs baseline: 1.0320x; 1.0320x over previous
import jax
import jax.numpy as jnp
from jax import lax
from jax.experimental import pallas as pl
from jax.experimental.pallas import tpu as pltpu
from jax.experimental.pallas import tpu_sc as plsc

B = 16384
T = 26
D = 16
BCW = 64

NC = 2
NS = 16
NW = NC * NS
BPW = B // NW
C = 64
NCHUNK = BPW // C
NG = C // 16


def _body(user_hbm, item_hbm, uemb_hbm, iemb_hbm, ubc_hbm, ibc_hbm,
          gb_hbm, gc_hbm, out_hbm,
          idx_u, idx_i, urows, irows, bcu, bci, gbv, gcv, outv, sem):
  wid = lax.axis_index("s") * NC + lax.axis_index("c")
  base = wid * BPW

  pltpu.sync_copy(user_hbm.at[pl.ds(base, BPW)], idx_u)
  pltpu.sync_copy(item_hbm.at[pl.ds(base, BPW)], idx_i)
  pltpu.sync_copy(gb_hbm, gbv)
  pltpu.sync_copy(gc_hbm, gcv)
  gb0 = gbv[pl.ds(0, 16)]
  gb1 = gbv[pl.ds(16, 16)]
  gc0 = gcv[pl.ds(0, 16)]
  gc1 = gcv[pl.ds(16, 16)]

  iota = lax.iota(jnp.int32, 16)
  rot = [jnp.bitwise_and(iota + d, 15) for d in range(D)]

  @pl.loop(0, NCHUNK)
  def _chunk(c):
    iu = idx_u.at[pl.ds(c * C, C)]
    ii = idx_i.at[pl.ds(c * C, C)]
    cps = [
        pltpu.async_copy(uemb_hbm.at[iu], urows, sem),
        pltpu.async_copy(iemb_hbm.at[ii], irows, sem),
        pltpu.async_copy(ubc_hbm.at[iu], bcu, sem),
        pltpu.async_copy(ibc_hbm.at[ii], bci, sem),
    ]
    for cp in cps:
      cp.wait()

    @pl.loop(0, NG)
    def _group(g):
      rvec = iota + g * 16
      acc = jnp.zeros((16,), jnp.float32)
      for t in range(T):
        tvec = jnp.full((16,), t, jnp.int32)
        ps = [jnp.zeros((16,), jnp.float32) for _ in range(4)]
        for d in range(D):
          uu = plsc.load_gather(urows, [rvec, tvec, rot[d]])
          iv = plsc.load_gather(irows, [rvec, tvec, rot[d]])
          ps[d % 4] = ps[d % 4] + uu * iv
        s = (ps[0] + ps[1]) + (ps[2] + ps[3])
        s = s + plsc.load_gather(bcu, [rvec, tvec])
        s = s + plsc.load_gather(bci, [rvec, tvec])
        s = s + (gb0[t] if t < 16 else gb1[t - 16])
        sig = 1.0 / (1.0 + jnp.exp(-s))
        ctvec = jnp.full((16,), T + t, jnp.int32)
        cf = plsc.load_gather(bcu, [rvec, ctvec])
        cf = cf + plsc.load_gather(bci, [rvec, ctvec])
        cf = cf + (gc0[t] if t < 16 else gc1[t - 16])
        acc = acc + sig * cf
      outv[pl.ds(c * C + g * 16, 16)] = acc

  pltpu.sync_copy(outv, out_hbm.at[pl.ds(base, BPW)])


def _concat_t_body(bt_ref, ct_ref, out_ref):
  blk = out_ref.shape[0]
  pad = jnp.zeros((blk, BCW - 2 * T), jnp.float32)
  out_ref[:, :] = jnp.concatenate(
      [bt_ref[:, :].T, ct_ref[:, :].T, pad], axis=1)


def _concat_t(bias_t, coeff_t):
  n = bias_t.shape[1]
  blk = 1024
  grid = (n + blk - 1) // blk
  return pl.pallas_call(
      _concat_t_body,
      grid=(grid,),
      in_specs=[
          pl.BlockSpec((T, blk), lambda j: (0, j)),
          pl.BlockSpec((T, blk), lambda j: (0, j)),
      ],
      out_specs=pl.BlockSpec((blk, BCW), lambda j: (j, 0)),
      out_shape=jax.ShapeDtypeStruct((n, BCW), jnp.float32),
  )(bias_t, coeff_t)


@jax.jit
def _run(user, item, uemb, iemb, ubc, ibc, gb, gc):
  mesh = plsc.VectorSubcoreMesh(core_axis_name="c", subcore_axis_name="s")
  f = pl.kernel(
      _body,
      out_type=jax.ShapeDtypeStruct((B,), jnp.float32),
      mesh=mesh,
      scratch_types=[
          pltpu.VMEM((BPW,), jnp.int32),
          pltpu.VMEM((BPW,), jnp.int32),
          pltpu.VMEM((C, T, D), jnp.float32),
          pltpu.VMEM((C, T, D), jnp.float32),
          pltpu.VMEM((C, BCW), jnp.float32),
          pltpu.VMEM((C, BCW), jnp.float32),
          pltpu.VMEM((32,), jnp.float32),
          pltpu.VMEM((32,), jnp.float32),
          pltpu.VMEM((BPW,), jnp.float32),
          pltpu.SemaphoreType.DMA,
      ],
      compiler_params=pltpu.CompilerParams(use_tc_tiling_on_sc=False,
                                           needs_layout_passes=False),
  )
  return f(user, item, uemb, iemb, ubc, ibc, gb, gc)


def kernel(user, item, user_tag_embeddings, item_tag_embeddings,
           user_aspect_bias, item_aspect_bias, global_aspect_bias,
           user_coeff, item_coeff, global_coeff):
  user = user.astype(jnp.int32)
  item = item.astype(jnp.int32)
  ubc = _concat_t(user_aspect_bias.T, user_coeff.T)
  ibc = _concat_t(item_aspect_bias.T, item_coeff.T)
  gb = jnp.pad(global_aspect_bias.reshape(T), (0, 32 - T))
  gc = jnp.pad(global_coeff.reshape(T), (0, 32 - T))
  return _run(user, item, user_tag_embeddings, item_tag_embeddings,
              ubc, ibc, gb, gc)


# speedup vs baseline: 1.0511x; 1.0185x over previous
import jax
import jax.numpy as jnp
from jax import lax
from jax.experimental import pallas as pl
from jax.experimental.pallas import tpu as pltpu
from jax.experimental.pallas import tpu_sc as plsc

B = 16384
T = 26
D = 16

NC = 2
NS = 16
NW = NC * NS
BPW = B // NW
C = 64
NCHUNK = BPW // C
NG = C // 16


def _body(user_hbm, item_hbm, uemb_hbm, iemb_hbm, bsum_hbm, csum_hbm,
          out_hbm, idx_u, idx_i, urows, irows, bs, cs, outv, sem):
  wid = lax.axis_index("s") * NC + lax.axis_index("c")
  base = wid * BPW

  pltpu.sync_copy(user_hbm.at[pl.ds(base, BPW)], idx_u)
  pltpu.sync_copy(item_hbm.at[pl.ds(base, BPW)], idx_i)
  pltpu.sync_copy(bsum_hbm.at[pl.ds(base * T, BPW * T)], bs)
  pltpu.sync_copy(csum_hbm.at[pl.ds(base * T, BPW * T)], cs)

  iota = lax.iota(jnp.int32, 16)
  rot = [jnp.bitwise_and(iota + d, 15) for d in range(D)]

  @pl.loop(0, NCHUNK)
  def _chunk(c):
    iu = idx_u.at[pl.ds(c * C, C)]
    ii = idx_i.at[pl.ds(c * C, C)]
    cps = [
        pltpu.async_copy(uemb_hbm.at[iu], urows, sem),
        pltpu.async_copy(iemb_hbm.at[ii], irows, sem),
    ]
    for cp in cps:
      cp.wait()

    @pl.loop(0, NG)
    def _group(g):
      rvec = iota + g * 16
      fbase = (rvec + c * C) * T

      @pl.loop(0, T, init_carry=jnp.zeros((16,), jnp.float32), unroll=2)
      def _tag(t, acc):
        tvec = jnp.full((16,), t, jnp.int32)
        ps = [jnp.zeros((16,), jnp.float32) for _ in range(4)]
        for d in range(D):
          uu = plsc.load_gather(urows, [rvec, tvec, rot[d]])
          iv = plsc.load_gather(irows, [rvec, tvec, rot[d]])
          ps[d % 4] = ps[d % 4] + uu * iv
        s = (ps[0] + ps[1]) + (ps[2] + ps[3])
        s = s + plsc.load_gather(bs, [fbase + t])
        sig = 1.0 / (1.0 + jnp.exp(-s))
        cf = plsc.load_gather(cs, [fbase + t])
        return acc + sig * cf

      outv[pl.ds(c * C + g * 16, 16)] = _tag

  pltpu.sync_copy(outv, out_hbm.at[pl.ds(base, BPW)])


@jax.jit
def _run(user, item, uemb, iemb, bsum, csum):
  mesh = plsc.VectorSubcoreMesh(core_axis_name="c", subcore_axis_name="s")
  f = pl.kernel(
      _body,
      out_type=jax.ShapeDtypeStruct((B,), jnp.float32),
      mesh=mesh,
      scratch_types=[
          pltpu.VMEM((BPW,), jnp.int32),
          pltpu.VMEM((BPW,), jnp.int32),
          pltpu.VMEM((C, T, D), jnp.float32),
          pltpu.VMEM((C, T, D), jnp.float32),
          pltpu.VMEM((BPW * T,), jnp.float32),
          pltpu.VMEM((BPW * T,), jnp.float32),
          pltpu.VMEM((BPW,), jnp.float32),
          pltpu.SemaphoreType.DMA,
      ],
      compiler_params=pltpu.CompilerParams(use_tc_tiling_on_sc=False,
                                           needs_layout_passes=False),
  )
  return f(user, item, uemb, iemb, bsum, csum)


def kernel(user, item, user_tag_embeddings, item_tag_embeddings,
           user_aspect_bias, item_aspect_bias, global_aspect_bias,
           user_coeff, item_coeff, global_coeff):
  user = user.astype(jnp.int32)
  item = item.astype(jnp.int32)
  bsum = (jnp.take(user_aspect_bias, user, axis=0)
          + jnp.take(item_aspect_bias, item, axis=0)
          + global_aspect_bias).reshape(-1)
  csum = (jnp.take(user_coeff, user, axis=0)
          + jnp.take(item_coeff, item, axis=0)
          + global_coeff).reshape(-1)
  return _run(user, item, user_tag_embeddings, item_tag_embeddings,
              bsum, csum)
